# baseline (speedup 1.0000x reference)
import jax
import jax.numpy as jnp
from jax import lax
from jax.experimental import pallas as pl
from jax.experimental.pallas import tpu as pltpu
from jax.experimental.pallas import tpu_sc as plsc

F = 128
B = 4096

NC = 2
NS = 16
NW = NC * NS
BPW = B // NW
L = 16
GROUPS = BPW // L
CHUNKS = F // L


def _body(user_hbm, item_hbm, bi_hbm, gu_hbm, gi_hbm,
          xui_out, beta_out, gu_out, gi_out,
          idxu_v, idxi_v, guv, giv, biv, xuiv, pbuf,
          sem_u, sem_i, sem_b, sem_uo, sem_io):
    wid = lax.axis_index("s") * NC + lax.axis_index("c")
    base = wid * BPW

    cp_ju = pltpu.async_copy(user_hbm.at[pl.ds(base, BPW)], idxu_v, sem_u)
    cp_ji = pltpu.async_copy(item_hbm.at[pl.ds(base, BPW)], idxi_v, sem_i)
    cp_ju.wait()
    cp_u = pltpu.async_copy(gu_hbm.at[idxu_v], guv, sem_u)
    cp_ji.wait()
    cp_b = pltpu.async_copy(bi_hbm.at[idxi_v], biv, sem_b)
    cp_i = pltpu.async_copy(gi_hbm.at[idxi_v], giv, sem_i)

    cp_u.wait()
    cp_uo = pltpu.async_copy(guv, gu_out.at[pl.ds(base, BPW)], sem_uo)
    cp_i.wait()
    cp_io = pltpu.async_copy(giv, gi_out.at[pl.ds(base, BPW)], sem_io)
    cp_b.wait()
    cp_bo = pltpu.async_copy(biv, beta_out.at[pl.ds(base, BPW)], sem_b)

    def row(r, carry):
        p = guv[r, pl.ds(0, L)] * giv[r, pl.ds(0, L)]
        for c in range(1, CHUNKS):
            p = p + guv[r, pl.ds(c * L, L)] * giv[r, pl.ds(c * L, L)]
        pbuf[r, :] = p
        return carry

    lax.fori_loop(0, BPW, row, 0)

    lane = lax.iota(jnp.int32, L)

    def group(g, carry):
        rows = g * L + lane
        out = biv[pl.ds(g * L, L)]
        for j in range(L):
            col = jnp.full((L,), j, dtype=jnp.int32)
            out = out + plsc.load_gather(pbuf, [rows, col])
        xuiv[pl.ds(g * L, L)] = out
        return carry

    lax.fori_loop(0, GROUPS, group, 0)

    pltpu.sync_copy(xuiv, xui_out.at[pl.ds(base, BPW)])
    cp_uo.wait()
    cp_io.wait()
    cp_bo.wait()


@jax.jit
def _run(user, item, Bi, Gu, Gi):
    mesh = plsc.VectorSubcoreMesh(
        core_axis_name="c", subcore_axis_name="s",
        num_cores=NC, num_subcores=NS)
    out_type = (
        jax.ShapeDtypeStruct((B,), jnp.float32),
        jax.ShapeDtypeStruct((B,), jnp.float32),
        jax.ShapeDtypeStruct((B, F), jnp.float32),
        jax.ShapeDtypeStruct((B, F), jnp.float32),
    )
    scratch = [
        pltpu.VMEM((BPW,), jnp.int32),
        pltpu.VMEM((BPW,), jnp.int32),
        pltpu.VMEM((BPW, F), jnp.float32),
        pltpu.VMEM((BPW, F), jnp.float32),
        pltpu.VMEM((BPW,), jnp.float32),
        pltpu.VMEM((BPW,), jnp.float32),
        pltpu.VMEM((BPW, L), jnp.float32),
        pltpu.SemaphoreType.DMA,
        pltpu.SemaphoreType.DMA,
        pltpu.SemaphoreType.DMA,
        pltpu.SemaphoreType.DMA,
        pltpu.SemaphoreType.DMA,
    ]
    f = pl.kernel(_body, out_type=out_type, mesh=mesh,
                  scratch_types=scratch,
                  compiler_params=pltpu.CompilerParams(
                      needs_layout_passes=False))
    return f(user, item, Bi, Gu, Gi)


def kernel(user, item, Bi, Gu, Gi):
    xui, beta_i, gamma_u, gamma_i = _run(
        user.astype(jnp.int32), item.astype(jnp.int32), Bi, Gu, Gi)
    return (xui, beta_i, gamma_u, gamma_i)


# speedup vs baseline: 1.0032x; 1.0032x over previous
import jax
import jax.numpy as jnp
from jax import lax
from jax.experimental import pallas as pl
from jax.experimental.pallas import tpu as pltpu
from jax.experimental.pallas import tpu_sc as plsc

F = 128
B = 4096

NC = 2
NS = 16
NW = NC * NS
BPW = B // NW
L = 16
GROUPS = BPW // L
CHUNKS = F // L


def _body(user_hbm, item_hbm, bi_hbm, gu_hbm, gi_hbm,
          xui_out, beta_out, gu_out, gi_out,
          idxu_v, idxi_v, guv, giv, biv, xuiv, pbuf,
          sem_u, sem_i, sem_b, sem_uo, sem_io):
    wid = lax.axis_index("s") * NC + lax.axis_index("c")
    base = wid * BPW

    cp_ju = pltpu.async_copy(user_hbm.at[pl.ds(base, BPW)], idxu_v, sem_u)
    cp_ji = pltpu.async_copy(item_hbm.at[pl.ds(base, BPW)], idxi_v, sem_i)
    cp_ju.wait()
    cp_u = pltpu.async_copy(gu_hbm.at[idxu_v], guv, sem_u)
    cp_ji.wait()
    cp_b = pltpu.async_copy(bi_hbm.at[idxi_v], biv, sem_b)
    cp_i = pltpu.async_copy(gi_hbm.at[idxi_v], giv, sem_i)

    cp_u.wait()
    cp_uo = pltpu.async_copy(guv, gu_out.at[pl.ds(base, BPW)], sem_uo)
    cp_i.wait()
    cp_io = pltpu.async_copy(giv, gi_out.at[pl.ds(base, BPW)], sem_io)
    cp_b.wait()
    cp_bo = pltpu.async_copy(biv, beta_out.at[pl.ds(base, BPW)], sem_b)

    def row(r, carry):
        def chunk(c, p):
            return p + guv[r, pl.ds(c * L, L)] * giv[r, pl.ds(c * L, L)]
        pbuf[r, :] = lax.fori_loop(
            1, CHUNKS, chunk, guv[r, pl.ds(0, L)] * giv[r, pl.ds(0, L)])
        return carry

    lax.fori_loop(0, BPW, row, 0)

    lane = lax.iota(jnp.int32, L)

    def group(g, carry):
        rows = g * L + lane
        out = biv[pl.ds(g * L, L)]
        for j in range(L):
            col = jnp.full((L,), j, dtype=jnp.int32)
            out = out + plsc.load_gather(pbuf, [rows, col])
        xuiv[pl.ds(g * L, L)] = out
        return carry

    lax.fori_loop(0, GROUPS, group, 0)

    pltpu.sync_copy(xuiv, xui_out.at[pl.ds(base, BPW)])
    cp_uo.wait()
    cp_io.wait()
    cp_bo.wait()


@jax.jit
def _run(user, item, Bi, Gu, Gi):
    mesh = plsc.VectorSubcoreMesh(
        core_axis_name="c", subcore_axis_name="s",
        num_cores=NC, num_subcores=NS)
    out_type = (
        jax.ShapeDtypeStruct((B,), jnp.float32),
        jax.ShapeDtypeStruct((B,), jnp.float32),
        jax.ShapeDtypeStruct((B, F), jnp.float32),
        jax.ShapeDtypeStruct((B, F), jnp.float32),
    )
    scratch = [
        pltpu.VMEM((BPW,), jnp.int32),
        pltpu.VMEM((BPW,), jnp.int32),
        pltpu.VMEM((BPW, F), jnp.float32),
        pltpu.VMEM((BPW, F), jnp.float32),
        pltpu.VMEM((BPW,), jnp.float32),
        pltpu.VMEM((BPW,), jnp.float32),
        pltpu.VMEM((BPW, L), jnp.float32),
        pltpu.SemaphoreType.DMA,
        pltpu.SemaphoreType.DMA,
        pltpu.SemaphoreType.DMA,
        pltpu.SemaphoreType.DMA,
        pltpu.SemaphoreType.DMA,
    ]
    f = pl.kernel(_body, out_type=out_type, mesh=mesh,
                  scratch_types=scratch,
                  compiler_params=pltpu.CompilerParams(
                      needs_layout_passes=False))
    return f(user, item, Bi, Gu, Gi)


def kernel(user, item, Bi, Gu, Gi):
    xui, beta_i, gamma_u, gamma_i = _run(
        user.astype(jnp.int32), item.astype(jnp.int32), Bi, Gu, Gi)
    return (xui, beta_i, gamma_u, gamma_i)
